# baseline (speedup 1.0000x reference)
import functools

import jax
import jax.numpy as jnp
from jax import lax
from jax.experimental import pallas as pl
from jax.experimental.pallas import tpu as pltpu
from jax.experimental.pallas import tpu_sc as plsc

B, S = 1024, 50
VOCAB = 1000
ROWS = B * S
NC, NS, L = 2, 16, 16
NW = NC * NS
RPW = ROWS // NW
CH = 32
NCHUNK = RPW // CH

_mesh = plsc.VectorSubcoreMesh(core_axis_name="c", subcore_axis_name="s")


@functools.partial(
    pl.kernel,
    mesh=_mesh,
    out_type=jax.ShapeDtypeStruct((ROWS * VOCAB,), jnp.float32),
    scratch_types=[
        pltpu.VMEM((RPW,), jnp.int32),
        pltpu.VMEM((CH * VOCAB,), jnp.float32),
    ],
    compiler_params=pltpu.CompilerParams(needs_layout_passes=False),
)
def _onehot_sc(x_hbm, out_hbm, idx_v, buf):
    wid = lax.axis_index("s") * NC + lax.axis_index("c")
    base_row = wid * RPW
    pltpu.sync_copy(x_hbm.at[pl.ds(base_row, RPW)], idx_v)

    zeros = jnp.zeros((L,), jnp.float32)
    ones = jnp.ones((L,), jnp.float32)

    def zero_body(i, carry):
        buf[pl.ds(i * L, L)] = zeros
        return carry

    lax.fori_loop(0, CH * VOCAB // L, zero_body, 0)

    def offsets(c, g):
        idx_vec = idx_v[pl.ds(c * CH + g * L, L)]
        local_row = lax.iota(jnp.int32, L) + g * L
        return local_row * VOCAB + idx_vec

    def chunk_body(c, carry):
        for g in range(CH // L):
            plsc.store_scatter(buf, [offsets(c, g)], ones)
        dst = out_hbm.at[pl.ds((base_row + c * CH) * VOCAB, CH * VOCAB)]
        pltpu.sync_copy(buf, dst)
        for g in range(CH // L):
            plsc.store_scatter(buf, [offsets(c, g)], zeros)
        return carry

    lax.fori_loop(0, NCHUNK, chunk_body, 0)


def kernel(X):
    xflat = X.reshape(-1).astype(jnp.int32)
    out = _onehot_sc(xflat)
    return out.reshape(B, S, VOCAB)
